# baseline (speedup 1.0000x reference)
import functools

import jax
import jax.numpy as jnp
from jax import lax
from jax.experimental import pallas as pl
from jax.experimental.pallas import tpu as pltpu
from jax.experimental.pallas import tpu_sc as plsc

_INFO = plsc.get_sparse_core_info()
_NC, _NS = _INFO.num_cores, _INFO.num_subcores
_NW = _NC * _NS

_D = 32
_B = 16384 * 50
_BPW = _B // _NW
_NBUF = 4
_C = 800
_NCHUNK = _BPW // _C
_NGROUP = _NCHUNK // _NBUF


@functools.partial(
    pl.kernel,
    mesh=plsc.VectorSubcoreMesh(core_axis_name="c", subcore_axis_name="s"),
    out_type=jax.ShapeDtypeStruct((_B, _D), jnp.float32),
    scratch_types=[
        pltpu.VMEM((_BPW,), jnp.int32),
        pltpu.VMEM((_NBUF, _C, _D), jnp.float32),
        [pltpu.SemaphoreType.DMA] * _NBUF,
        [pltpu.SemaphoreType.DMA] * _NBUF,
    ],
    compiler_params=pltpu.CompilerParams(use_tc_tiling_on_sc=False),
)
def _gather_kernel(table_hbm, idx_hbm, out_hbm, idx_v, rows_v, gsems, psems):
    wid = lax.axis_index("s") * _NC + lax.axis_index("c")
    base = wid * _BPW
    pltpu.sync_copy(idx_hbm.at[pl.ds(base, _BPW)], idx_v)

    def gather(j, b):
        return pltpu.make_async_copy(
            table_hbm.at[idx_v.at[pl.ds(j * _C, _C)]], rows_v.at[b], gsems[b]
        )

    def put(j, b):
        return pltpu.make_async_copy(
            rows_v.at[b], out_hbm.at[pl.ds(base + j * _C, _C)], psems[b]
        )

    @pl.loop(0, _NGROUP)
    def _group(g):
        for b in range(_NBUF):
            @pl.when(g > 0)
            def _():
                put(g * _NBUF + b, b).wait()
            gather(g * _NBUF + b, b).start()
        for b in range(_NBUF):
            gather(g * _NBUF + b, b).wait()
            put(g * _NBUF + b, b).start()

    for b in range(_NBUF):
        put((_NGROUP - 1) * _NBUF + b, b).wait()


def kernel(topic_ids, table):
    ids = topic_ids.reshape(-1).astype(jnp.int32)
    out = _gather_kernel(table, ids)
    return out.reshape(topic_ids.shape + (_D,))


# speedup vs baseline: 1.4871x; 1.4871x over previous
import functools

import numpy as np

import jax
import jax.numpy as jnp
from jax import lax
from jax.experimental import pallas as pl
from jax.experimental.pallas import tpu as pltpu
from jax.experimental.pallas import tpu_sc as plsc

_INFO = plsc.get_sparse_core_info()
_NC, _NS = _INFO.num_cores, _INFO.num_subcores
_NW = _NC * _NS

_D = 32
_NB = 16384
_NH = 50
_BT = _NB // 128
_UNITS = _NH * _BT
_UPW = _UNITS // _NW
_IPW = _UPW * 128
_NBUF = 2


@functools.partial(
    pl.kernel,
    mesh=plsc.VectorSubcoreMesh(core_axis_name="c", subcore_axis_name="s"),
    out_type=jax.ShapeDtypeStruct((_NH, 4, 128, 1024), jnp.float32),
    scratch_types=[
        pltpu.VMEM((_IPW,), jnp.int32),
        pltpu.VMEM((_NBUF, 128, _D), jnp.float32),
        pltpu.VMEM((_NBUF, 4096), jnp.float32),
        pltpu.VMEM((128, _D), jnp.int32),
        pltpu.VMEM((_NBUF, 16), jnp.int32),
        [pltpu.SemaphoreType.DMA] * _NBUF,
        [pltpu.SemaphoreType.DMA] * _NBUF,
    ],
    compiler_params=pltpu.CompilerParams(
        use_tc_tiling_on_sc=False, needs_layout_passes=False),
)
def _gather_kernel(table_hbm, ids_hbm, offs_hbm, rows_hbm, out_hbm,
                   idxs, gbuf, tbuf, offs_v, rows_v, gsems, osems):
    wid = lax.axis_index("s") * _NC + lax.axis_index("c")
    u0 = wid * _UPW
    pltpu.sync_copy(ids_hbm.at[pl.ds(u0 * 128, _IPW)], idxs)
    pltpu.sync_copy(offs_hbm, offs_v)
    pltpu.sync_copy(rows_hbm, rows_v)

    def gather(i, s):
        return pltpu.make_async_copy(
            table_hbm.at[idxs.at[pl.ds(i * 128, 128)]], gbuf.at[s], gsems[s])

    def ocopies(i, s):
        u = u0 + i
        return [
            pltpu.make_async_copy(
                tbuf.at[s, pl.ds(dt * 1024, 1024)],
                out_hbm.at[u // _BT, dt, u % _BT], osems[s])
            for dt in range(4)
        ]

    def transform(s):
        row = rows_v[s, pl.ds(0, 16)]
        for b in range(128):
            lo = gbuf[s, b, pl.ds(0, 16)]
            hi = gbuf[s, b, pl.ds(16, 16)]
            alo = offs_v[b, pl.ds(0, 16)]
            ahi = offs_v[b, pl.ds(16, 16)]
            plsc.store_scatter(tbuf, [row, alo], lo)
            plsc.store_scatter(tbuf, [row, ahi], hi)

    gather(0, 0).start()

    @pl.loop(0, _UPW // _NBUF)
    def _step(g):
        for b in range(_NBUF):
            i = g * _NBUF + b
            s = b

            @pl.when(i + 1 < _UPW)
            def _():
                gather(i + 1, 1 - s).start()
            gather(i, s).wait()

            @pl.when(i >= _NBUF)
            def _():
                for c in ocopies(i - _NBUF, s):
                    c.wait()
            transform(s)
            for c in ocopies(i, s):
                c.start()

    for c in ocopies(_UPW - 2, 0):
        c.wait()
    for c in ocopies(_UPW - 1, 1):
        c.wait()


_OFFS = np.array(
    [[(d // 8) * 1024 + (d % 8) * 128 + b for d in range(_D)]
     for b in range(128)], dtype=np.int32)
_ROWS = np.repeat(np.arange(_NBUF, dtype=np.int32)[:, None], 16, axis=1)


def kernel(topic_ids, table):
    ids = topic_ids.T.reshape(-1).astype(jnp.int32)
    out4 = _gather_kernel(table, ids, jnp.asarray(_OFFS), jnp.asarray(_ROWS))
    out5 = out4.reshape(_NH, 4, 128, 8, 128)
    return jnp.transpose(out5, (2, 4, 0, 1, 3)).reshape(_NB, _NH, _D)
